# baseline (speedup 1.0000x reference)
import jax
import jax.numpy as jnp
from jax.experimental import pallas as pl

_RES = (480, 640)
_FLOW_SCALING = 640.0


def _div_body(iwe_ref, cnt_ref, out_ref):
    iwe = iwe_ref[...]
    cnt = cnt_ref[...]
    out_ref[...] = jnp.where(cnt > 0, iwe / jnp.where(cnt > 0, cnt, 1.0), iwe)


def kernel(flow, event_list, pol_mask):
    res = _RES
    B = flow.shape[0]
    hw = res[0] * res[1]

    lin = event_list[:, :, 1] * res[1] + event_list[:, :, 2]
    flow_idx = lin.astype(jnp.int64)
    flow_flat = flow.reshape(B, 2, -1)
    efy = jnp.take_along_axis(flow_flat[:, 1, :], flow_idx, axis=1)
    efx = jnp.take_along_axis(flow_flat[:, 0, :], flow_idx, axis=1)
    event_flow = jnp.stack([efy, efx], axis=2)

    warped = event_list[:, :, 1:3] + (1.0 - event_list[:, :, 0:1]) * event_flow * _FLOW_SCALING
    idx = jnp.round(warped)
    mask = ((idx[:, :, 0:1] >= 0) & (idx[:, :, 0:1] < res[0]) &
            (idx[:, :, 1:2] >= 0) & (idx[:, :, 1:2] < res[1])).astype(idx.dtype)
    idx = idx * mask
    fw_weights = mask
    fw_idx = idx[:, :, 0:1] * res[1] + idx[:, :, 1:2]
    fw_idx_i = fw_idx.astype(jnp.int64)

    def _interp(weights):
        iwe = jnp.zeros((B, hw), dtype=jnp.float32)
        iwe = iwe.at[jnp.arange(B)[:, None], fw_idx_i[:, :, 0]].add(weights[:, :, 0])
        return iwe.reshape(B, 1, res[0], res[1])

    iwe_pos = _interp(fw_weights * pol_mask[:, :, 0:1])
    iwe_neg = _interp(fw_weights * pol_mask[:, :, 1:2])

    pol_list = event_list[:, :, 3:4]
    pol_list = jnp.where(pol_list < 1, 0.0, pol_list)
    pol_list = jnp.where(fw_weights == 0, 2.0, pol_list)
    idx0 = lin.astype(jnp.int64)[:, :, None]
    m_idx = idx0 * hw + fw_idx_i
    pm_idx = pol_list.astype(jnp.int64) * (hw * hw) + m_idx
    pos_list, neg_list = [], []
    for b in range(B):
        s_pm = jnp.sort(pm_idx[b, :, 0])
        is_first = jnp.concatenate([jnp.ones((1,), dtype=bool), s_pm[1:] != s_pm[:-1]])
        pol = s_pm // (hw * hw)
        fwpix = (s_pm % (hw * hw)) % hw
        mask_pos = jnp.where(pol == 2, 0, pol).astype(jnp.float32)
        mask_neg = (1 - jnp.where(pol == 2, 1, pol)).astype(jnp.float32)
        cf = is_first.astype(jnp.float32)
        pos_list.append(jnp.zeros((hw,), jnp.float32).at[fwpix].add(mask_pos * cf))
        neg_list.append(jnp.zeros((hw,), jnp.float32).at[fwpix].add(mask_neg * cf))
    pos_c = jnp.stack(pos_list).reshape(B, 1, res[0], res[1])
    neg_c = jnp.stack(neg_list).reshape(B, 1, res[0], res[1])

    iwe = jnp.concatenate([iwe_pos, iwe_neg], axis=1)
    cnt = jnp.concatenate([pos_c, neg_c], axis=1)

    out = pl.pallas_call(
        _div_body,
        out_shape=jax.ShapeDtypeStruct(iwe.shape, iwe.dtype),
    )(iwe, cnt)
    return out


# speedup vs baseline: 1.2539x; 1.2539x over previous
import jax
import jax.numpy as jnp
import numpy as np
from jax.experimental import pallas as pl

_RES = (480, 640)
_HW = _RES[0] * _RES[1]
_FLOW_SCALING = 640.0
_I0 = np.int32(0)


def _warp_body(ts_ref, y_ref, x_ref, p_ref, efy_ref, efx_ref,
               fw_ref, wpos_ref, wneg_ref, ka_ref):
    ts = ts_ref[...]
    y = y_ref[...]
    x = x_ref[...]
    p = p_ref[...]
    dt = 1.0 - ts
    wy = y + dt * efy_ref[...] * _FLOW_SCALING
    wx = x + dt * efx_ref[...] * _FLOW_SCALING
    iy = jnp.round(wy)
    ix = jnp.round(wx)
    m = (iy >= 0) & (iy < _RES[0]) & (ix >= 0) & (ix < _RES[1])
    iy = jnp.where(m, iy, 0.0)
    ix = jnp.where(m, ix, 0.0)
    fw = (iy * _RES[1] + ix).astype(jnp.int32)
    w = m.astype(jnp.float32)
    wpos_ref[...] = w * (p > 0).astype(jnp.float32)
    wneg_ref[...] = w * (p < 0).astype(jnp.float32)
    lin = (y * _RES[1] + x).astype(jnp.int32)
    pol = (p > 0).astype(jnp.int32)
    pol = jnp.where(m, pol, jnp.int32(2))
    fw_ref[...] = fw
    ka_ref[...] = pol * _HW + lin


def _uniq_body(ka_ref, kb_ref, pka_ref, pkb_ref, cpos_ref, cneg_ref):
    ka = ka_ref[...]
    kb = kb_ref[...]
    is_first = (ka != pka_ref[...]) | (kb != pkb_ref[...])
    cf = is_first.astype(jnp.float32)
    cpos_ref[...] = cf * ((ka >= _HW) & (ka < 2 * _HW)).astype(jnp.float32)
    cneg_ref[...] = cf * (ka < _HW).astype(jnp.float32)


def _div_body(iwe_ref, cnt_ref, out_ref):
    iwe = iwe_ref[...]
    cnt = cnt_ref[...]
    out_ref[...] = jnp.where(cnt > 0, iwe / jnp.where(cnt > 0, cnt, 1.0), iwe)


def kernel(flow, event_list, pol_mask):
    B, N = event_list.shape[0], event_list.shape[1]
    flow = flow.astype(jnp.float32)
    event_list = event_list.astype(jnp.float32)

    ts = event_list[:, :, 0]
    y = event_list[:, :, 1]
    x = event_list[:, :, 2]
    p = event_list[:, :, 3]

    lin = (y * _RES[1] + x).astype(jnp.int32)
    flow_flat = flow.reshape(B, 2, _HW)
    efy = jnp.take_along_axis(flow_flat[:, 1, :], lin, axis=1)
    efx = jnp.take_along_axis(flow_flat[:, 0, :], lin, axis=1)

    NR = N // 128
    row = pl.BlockSpec((1, NR, 128), lambda b: (b, _I0, _I0))
    r3 = lambda a: a.reshape(B, NR, 128)
    fw, wpos, wneg, ka = pl.pallas_call(
        _warp_body,
        grid=(B,),
        in_specs=[row] * 6,
        out_specs=[row] * 4,
        out_shape=[
            jax.ShapeDtypeStruct((B, NR, 128), jnp.int32),
            jax.ShapeDtypeStruct((B, NR, 128), jnp.float32),
            jax.ShapeDtypeStruct((B, NR, 128), jnp.float32),
            jax.ShapeDtypeStruct((B, NR, 128), jnp.int32),
        ],
    )(r3(ts), r3(y), r3(x), r3(p), r3(efy), r3(efx))
    fw = fw.reshape(B, N)
    wpos = wpos.reshape(B, N)
    wneg = wneg.reshape(B, N)
    ka = ka.reshape(B, N)

    ka_s, kb_s = jax.lax.sort((ka, kb := fw), dimension=1, num_keys=2)
    pad = jnp.full((B, 1), -1, jnp.int32)
    pka = jnp.concatenate([pad, ka_s[:, :-1]], axis=1)
    pkb = jnp.concatenate([pad, kb_s[:, :-1]], axis=1)

    cpos, cneg = pl.pallas_call(
        _uniq_body,
        grid=(B,),
        in_specs=[row] * 4,
        out_specs=[row] * 2,
        out_shape=[
            jax.ShapeDtypeStruct((B, NR, 128), jnp.float32),
            jax.ShapeDtypeStruct((B, NR, 128), jnp.float32),
        ],
    )(r3(ka_s), r3(kb_s), r3(pka), r3(pkb))
    cpos = cpos.reshape(B, N)
    cneg = cneg.reshape(B, N)

    bidx = jnp.arange(B)[:, None]
    zeros = jnp.zeros((B, _HW), jnp.float32)
    iwe_pos = zeros.at[bidx, fw].add(wpos)
    iwe_neg = zeros.at[bidx, fw].add(wneg)
    cnt_pos = zeros.at[bidx, kb_s].add(cpos)
    cnt_neg = zeros.at[bidx, kb_s].add(cneg)

    iwe = jnp.stack([iwe_pos, iwe_neg], axis=1).reshape(B, 2, _RES[0], _RES[1])
    cnt = jnp.stack([cnt_pos, cnt_neg], axis=1).reshape(B, 2, _RES[0], _RES[1])

    out = pl.pallas_call(
        _div_body,
        grid=(B,),
        in_specs=[pl.BlockSpec((1, 2, _RES[0], _RES[1]), lambda b: (b, _I0, _I0, _I0))] * 2,
        out_specs=pl.BlockSpec((1, 2, _RES[0], _RES[1]), lambda b: (b, _I0, _I0, _I0)),
        out_shape=jax.ShapeDtypeStruct(iwe.shape, jnp.float32),
    )(iwe, cnt)
    return out
